# baseline (speedup 1.0000x reference)
import functools

import jax
import jax.numpy as jnp
from jax import lax
from jax.experimental import pallas as pl
from jax.experimental.pallas import tpu as pltpu
from jax.experimental.pallas import tpu_sc as plsc

B, Q, T, H, W, P = 4, 300, 100, 128, 128, 12544
HW = H * W
NTILES = 32
TILES_PER_B = NTILES // B
IMGS_PER_B = Q + T
IMGS_PER_TILE = IMGS_PER_B // TILES_PER_B
LANES = 16
STEPS = P // LANES
PBLK = 1792
NPBLK = P // PBLK
VREGS_PER_BLK = PBLK // 128


def _prep_body(cx_ref, cy_ref, ipk_ref, w00_ref, w01_ref, w10_ref,
               w11_ref):
    cx = cx_ref[...]
    cy = cy_ref[...]
    gx = 2.0 * cx - 1.0
    gy = 2.0 * cy - 1.0
    x = ((gx + 1.0) * W - 1.0) / 2.0
    y = ((gy + 1.0) * H - 1.0) / 2.0
    x0 = jnp.floor(x)
    y0 = jnp.floor(y)
    x1 = x0 + 1.0
    y1 = y0 + 1.0
    wx1 = x - x0
    wx0 = 1.0 - wx1
    wy1 = y - y0
    wy0 = 1.0 - wy1

    def vmask(xi, yi):
        v = (xi >= 0) & (xi <= W - 1) & (yi >= 0) & (yi <= H - 1)
        return v.astype(jnp.float32)

    xc0 = jnp.clip(x0, 0, W - 1).astype(jnp.int32)
    xc1 = jnp.clip(x1, 0, W - 1).astype(jnp.int32)
    yc0 = jnp.clip(y0, 0, H - 1).astype(jnp.int32)
    yc1 = jnp.clip(y1, 0, H - 1).astype(jnp.int32)

    i00 = yc0 * W + xc0
    dx = xc1 - xc0
    dy = yc1 - yc0
    ipk_ref[...] = i00 | (dx << 14) | (dy << 15)
    w00_ref[...] = (wy0 * wx0) * vmask(x0, y0)
    w01_ref[...] = (wy0 * wx1) * vmask(x1, y0)
    w10_ref[...] = (wy1 * wx0) * vmask(x0, y1)
    w11_ref[...] = (wy1 * wx1) * vmask(x1, y1)


def _prep(cx, cy):
    out = (
        jax.ShapeDtypeStruct((B, P), jnp.int32),
        jax.ShapeDtypeStruct((B, P), jnp.float32),
        jax.ShapeDtypeStruct((B, P), jnp.float32),
        jax.ShapeDtypeStruct((B, P), jnp.float32),
        jax.ShapeDtypeStruct((B, P), jnp.float32),
    )
    return pl.pallas_call(_prep_body, out_shape=out)(cx, cy)


def _sc_body(predflat, tgtflat, ipk_h, w00_h, w01_h, w10_h, w11_h,
             outp, outt, ipk_v, w00_v, w01_v, w10_v, w11_v, img_a, img_b,
             out_a, out_b, sem_ia, sem_ib, sem_oa, sem_ob):
    wid = lax.axis_index("c") * 16 + lax.axis_index("s")
    b = wid // TILES_PER_B
    t8 = wid % TILES_PER_B
    g0 = t8 * IMGS_PER_TILE

    pltpu.sync_copy(ipk_h.at[b], ipk_v)
    pltpu.sync_copy(w00_h.at[b], w00_v)
    pltpu.sync_copy(w01_h.at[b], w01_v)
    pltpu.sync_copy(w10_h.at[b], w10_v)
    pltpu.sync_copy(w11_h.at[b], w11_v)

    def start_img(g, img_v, sem):
        is_pred = g < Q

        @pl.when(is_pred)
        def _():
            pltpu.async_copy(predflat.at[b * Q + g], img_v, sem)

        @pl.when(jnp.logical_not(is_pred))
        def _():
            pltpu.async_copy(tgtflat.at[b * T + (g - Q)], img_v, sem)

    def wait_img(img_v, sem):
        pltpu.make_async_copy(predflat.at[0], img_v, sem).wait()

    def start_out(g, out_v, sem):
        is_pred = g < Q

        @pl.when(is_pred)
        def _():
            pltpu.async_copy(out_v, outp.at[b * Q + g], sem)

        @pl.when(jnp.logical_not(is_pred))
        def _():
            pltpu.async_copy(out_v, outt.at[b * T + (g - Q)], sem)

    def wait_out(out_v, sem):
        pltpu.make_async_copy(out_v, outp.at[0], sem).wait()

    def compute_pair():
        @plsc.parallel_loop(0, P, LANES, unroll=8)
        def _(i):
            sl = pl.ds(i, LANES)
            ipk = ipk_v[sl]
            i00 = ipk & 0x3FFF
            dx = lax.shift_right_logical(ipk, 14) & 1
            dy128 = (lax.shift_right_logical(ipk, 15) & 1) << 7
            i01 = i00 + dx
            i10 = i00 + dy128
            i11 = i10 + dx
            w00 = w00_v[sl]
            w01 = w01_v[sl]
            w10 = w10_v[sl]
            w11 = w11_v[sl]
            aA = plsc.load_gather(img_a, [i00])
            aB = plsc.load_gather(img_a, [i01])
            aC = plsc.load_gather(img_a, [i10])
            aD = plsc.load_gather(img_a, [i11])
            out_a[sl] = ((aA * w00 + aB * w01) + aC * w10) + aD * w11
            bA = plsc.load_gather(img_b, [i00])
            bB = plsc.load_gather(img_b, [i01])
            bC = plsc.load_gather(img_b, [i10])
            bD = plsc.load_gather(img_b, [i11])
            out_b[sl] = ((bA * w00 + bB * w01) + bC * w10) + bD * w11

    start_img(g0, img_a, sem_ia)
    start_img(g0 + 1, img_b, sem_ib)

    def pair(j, carry):
        ga = g0 + 2 * j
        wait_img(img_a, sem_ia)
        wait_img(img_b, sem_ib)

        @pl.when(j > 0)
        def _():
            wait_out(out_a, sem_oa)
            wait_out(out_b, sem_ob)

        compute_pair()
        start_out(ga, out_a, sem_oa)
        start_out(ga + 1, out_b, sem_ob)

        @pl.when(j < IMGS_PER_TILE // 2 - 1)
        def _():
            start_img(ga + 2, img_a, sem_ia)
            start_img(ga + 3, img_b, sem_ib)

        return carry

    lax.fori_loop(0, IMGS_PER_TILE // 2, pair, 0, unroll=False)
    wait_out(out_a, sem_oa)
    wait_out(out_b, sem_ob)


def _sc_sample(predflat, tgtflat, ipk, w00, w01, w10, w11):
    mesh = plsc.VectorSubcoreMesh(core_axis_name="c", subcore_axis_name="s")
    fn = pl.kernel(
        _sc_body,
        out_type=(
            jax.ShapeDtypeStruct((B * Q, P), jnp.float32),
            jax.ShapeDtypeStruct((B * T, P), jnp.float32),
        ),
        mesh=mesh,
        compiler_params=pltpu.CompilerParams(needs_layout_passes=False),
        scratch_types=[
            pltpu.VMEM((P,), jnp.int32),
            pltpu.VMEM((P,), jnp.float32),
            pltpu.VMEM((P,), jnp.float32),
            pltpu.VMEM((P,), jnp.float32),
            pltpu.VMEM((P,), jnp.float32),
            pltpu.VMEM((HW,), jnp.float32),
            pltpu.VMEM((HW,), jnp.float32),
            pltpu.VMEM((P,), jnp.float32),
            pltpu.VMEM((P,), jnp.float32),
            pltpu.SemaphoreType.DMA,
            pltpu.SemaphoreType.DMA,
            pltpu.SemaphoreType.DMA,
            pltpu.SemaphoreType.DMA,
        ],
    )
    return fn(predflat, tgtflat, ipk, w00, w01, w10, w11)


def _cost_body(pred_ref, tgt_ref, minv_ref, mini_ref, num_acc, ssum_acc,
               tsum_acc):
    pj = pl.program_id(1)

    @pl.when(pj == 0)
    def _():
        num_acc[...] = jnp.zeros_like(num_acc)
        ssum_acc[...] = jnp.zeros_like(ssum_acc)
        tsum_acc[...] = jnp.zeros_like(tsum_acc)

    sig = 1.0 / (1.0 + jnp.exp(-pred_ref[0]))
    tgt = tgt_ref[0]
    num_acc[...] += lax.dot_general(
        sig.astype(jnp.bfloat16), tgt.astype(jnp.bfloat16),
        (((1,), (1,)), ((), ())),
        preferred_element_type=jnp.float32)
    s_part = ssum_acc[...]
    t_part = tsum_acc[...]
    for k in range(VREGS_PER_BLK):
        s_part = s_part + sig[:, k * 128:(k + 1) * 128]
        t_part = t_part + tgt[:, k * 128:(k + 1) * 128]
    ssum_acc[...] = s_part
    tsum_acc[...] = t_part

    @pl.when(pj == NPBLK - 1)
    def _():
        num = num_acc[...]
        ssum = jnp.sum(ssum_acc[...], axis=1)
        tsum = jnp.sum(tsum_acc[...], axis=1)
        den = ssum[:, None] + tsum[None, :]
        cost = 1.0 - (2.0 * num + 1.0) / (den + 1.0)
        minv = jnp.min(cost, axis=1)
        iota = lax.broadcasted_iota(jnp.int32, (Q, T), 1)
        mini = jnp.min(jnp.where(cost == minv[:, None], iota, T), axis=1)
        minv_ref[0, 0] = minv
        mini_ref[0, 0] = mini.astype(jnp.int32)


def _cost(sampled_pred, sampled_tgt):
    grid = (B, NPBLK)
    out = pl.pallas_call(
        _cost_body,
        grid=grid,
        in_specs=[
            pl.BlockSpec((1, Q, PBLK), lambda b, pj: (b, 0, pj)),
            pl.BlockSpec((1, T, PBLK), lambda b, pj: (b, 0, pj)),
        ],
        out_specs=[
            pl.BlockSpec((1, 1, Q), lambda b, pj: (b, 0, 0)),
            pl.BlockSpec((1, 1, Q), lambda b, pj: (b, 0, 0)),
        ],
        out_shape=[
            jax.ShapeDtypeStruct((B, 1, Q), jnp.float32),
            jax.ShapeDtypeStruct((B, 1, Q), jnp.int32),
        ],
        scratch_shapes=[
            pltpu.VMEM((Q, T), jnp.float32),
            pltpu.VMEM((Q, 128), jnp.float32),
            pltpu.VMEM((T, 128), jnp.float32),
        ],
        compiler_params=pltpu.CompilerParams(
            dimension_semantics=("arbitrary", "arbitrary")),
    )(sampled_pred, sampled_tgt)
    return out


def kernel(pred_logits, pred_masks, tgt_masks, point_coords):
    del pred_logits
    cx = point_coords[..., 0]
    cy = point_coords[..., 1]
    ipk, w00, w01, w10, w11 = _prep(cx, cy)

    predflat = pred_masks.reshape(B * Q, HW)
    tgtflat = tgt_masks.reshape(B * T, HW)
    sp, st = _sc_sample(predflat, tgtflat, ipk, w00, w01, w10, w11)

    minv, mini = _cost(sp.reshape(B, Q, P), st.reshape(B, T, P))
    minv = minv.reshape(B, Q)
    mini = mini.reshape(B, Q)
    valid = minv < 0.4
    return minv, mini, valid


# speedup vs baseline: 1.1371x; 1.1371x over previous
import functools

import jax
import jax.numpy as jnp
from jax import lax
from jax.experimental import pallas as pl
from jax.experimental.pallas import tpu as pltpu
from jax.experimental.pallas import tpu_sc as plsc

B, Q, T, H, W, P = 4, 300, 100, 128, 128, 12544
HW = H * W
NTILES = 32
TILES_PER_B = NTILES // B
IMGS_PER_B = Q + T
IMGS_PER_TILE = IMGS_PER_B // TILES_PER_B
LANES = 16
STEPS = P // LANES
PBLK = 1792
NPBLK = P // PBLK
VREGS_PER_BLK = PBLK // 128


def _prep_body(cx_ref, cy_ref, ipk_ref, w00_ref, w01_ref, w10_ref,
               w11_ref):
    cx = cx_ref[...]
    cy = cy_ref[...]
    gx = 2.0 * cx - 1.0
    gy = 2.0 * cy - 1.0
    x = ((gx + 1.0) * W - 1.0) / 2.0
    y = ((gy + 1.0) * H - 1.0) / 2.0
    x0 = jnp.floor(x)
    y0 = jnp.floor(y)
    x1 = x0 + 1.0
    y1 = y0 + 1.0
    wx1 = x - x0
    wx0 = 1.0 - wx1
    wy1 = y - y0
    wy0 = 1.0 - wy1

    def vmask(xi, yi):
        v = (xi >= 0) & (xi <= W - 1) & (yi >= 0) & (yi <= H - 1)
        return v.astype(jnp.float32)

    xc0 = jnp.clip(x0, 0, W - 1).astype(jnp.int32)
    xc1 = jnp.clip(x1, 0, W - 1).astype(jnp.int32)
    yc0 = jnp.clip(y0, 0, H - 1).astype(jnp.int32)
    yc1 = jnp.clip(y1, 0, H - 1).astype(jnp.int32)

    i00 = yc0 * W + xc0
    dx = xc1 - xc0
    dy = yc1 - yc0
    ipk_ref[...] = i00 | (dx << 14) | (dy << 15)
    w00_ref[...] = (wy0 * wx0) * vmask(x0, y0)
    w01_ref[...] = (wy0 * wx1) * vmask(x1, y0)
    w10_ref[...] = (wy1 * wx0) * vmask(x0, y1)
    w11_ref[...] = (wy1 * wx1) * vmask(x1, y1)


def _prep(cx, cy):
    out = (
        jax.ShapeDtypeStruct((B, P), jnp.int32),
        jax.ShapeDtypeStruct((B, P), jnp.float32),
        jax.ShapeDtypeStruct((B, P), jnp.float32),
        jax.ShapeDtypeStruct((B, P), jnp.float32),
        jax.ShapeDtypeStruct((B, P), jnp.float32),
    )
    return pl.pallas_call(_prep_body, out_shape=out)(cx, cy)


def _sc_body(predflat, tgtflat, ipk_h, w00_h, w01_h, w10_h, w11_h,
             outp, outt, ipk_v, w00_v, w01_v, w10_v, w11_v, img_a, img_b,
             out_a, out_b, sem_ia, sem_ib, sem_oa, sem_ob):
    wid = lax.axis_index("c") * 16 + lax.axis_index("s")
    b = wid // TILES_PER_B
    t8 = wid % TILES_PER_B
    g0 = t8 * IMGS_PER_TILE

    pltpu.sync_copy(ipk_h.at[b], ipk_v)
    pltpu.sync_copy(w00_h.at[b], w00_v)
    pltpu.sync_copy(w01_h.at[b], w01_v)
    pltpu.sync_copy(w10_h.at[b], w10_v)
    pltpu.sync_copy(w11_h.at[b], w11_v)

    def start_img(g, img_v, sem):
        is_pred = g < Q

        @pl.when(is_pred)
        def _():
            pltpu.async_copy(predflat.at[b * Q + g], img_v, sem)

        @pl.when(jnp.logical_not(is_pred))
        def _():
            pltpu.async_copy(tgtflat.at[b * T + (g - Q)], img_v, sem)

    def wait_img(img_v, sem):
        pltpu.make_async_copy(predflat.at[0], img_v, sem).wait()

    def start_out(g, out_v, sem):
        is_pred = g < Q

        @pl.when(is_pred)
        def _():
            pltpu.async_copy(out_v, outp.at[b * Q + g], sem)

        @pl.when(jnp.logical_not(is_pred))
        def _():
            pltpu.async_copy(out_v, outt.at[b * T + (g - Q)], sem)

    def wait_out(out_v, sem):
        pltpu.make_async_copy(out_v, outp.at[0], sem).wait()

    def compute_pair():
        @plsc.parallel_loop(0, P, LANES, unroll=4)
        def _(i):
            sl = pl.ds(i, LANES)
            ipk = ipk_v[sl]
            i00 = ipk & 0x3FFF
            dx = lax.shift_right_logical(ipk, 14) & 1
            dy128 = (lax.shift_right_logical(ipk, 15) & 1) << 7
            i01 = i00 + dx
            i10 = i00 + dy128
            i11 = i10 + dx
            w00 = w00_v[sl]
            w01 = w01_v[sl]
            w10 = w10_v[sl]
            w11 = w11_v[sl]
            aA = plsc.load_gather(img_a, [i00])
            aB = plsc.load_gather(img_a, [i01])
            aC = plsc.load_gather(img_a, [i10])
            aD = plsc.load_gather(img_a, [i11])
            out_a[sl] = ((aA * w00 + aB * w01) + aC * w10) + aD * w11
            bA = plsc.load_gather(img_b, [i00])
            bB = plsc.load_gather(img_b, [i01])
            bC = plsc.load_gather(img_b, [i10])
            bD = plsc.load_gather(img_b, [i11])
            out_b[sl] = ((bA * w00 + bB * w01) + bC * w10) + bD * w11

    start_img(g0, img_a, sem_ia)
    start_img(g0 + 1, img_b, sem_ib)

    def pair(j, carry):
        ga = g0 + 2 * j
        wait_img(img_a, sem_ia)
        wait_img(img_b, sem_ib)

        @pl.when(j > 0)
        def _():
            wait_out(out_a, sem_oa)
            wait_out(out_b, sem_ob)

        compute_pair()
        start_out(ga, out_a, sem_oa)
        start_out(ga + 1, out_b, sem_ob)

        @pl.when(j < IMGS_PER_TILE // 2 - 1)
        def _():
            start_img(ga + 2, img_a, sem_ia)
            start_img(ga + 3, img_b, sem_ib)

        return carry

    lax.fori_loop(0, IMGS_PER_TILE // 2, pair, 0, unroll=False)
    wait_out(out_a, sem_oa)
    wait_out(out_b, sem_ob)


def _sc_sample(predflat, tgtflat, ipk, w00, w01, w10, w11):
    mesh = plsc.VectorSubcoreMesh(core_axis_name="c", subcore_axis_name="s")
    fn = pl.kernel(
        _sc_body,
        out_type=(
            jax.ShapeDtypeStruct((B * Q, P), jnp.float32),
            jax.ShapeDtypeStruct((B * T, P), jnp.float32),
        ),
        mesh=mesh,
        compiler_params=pltpu.CompilerParams(needs_layout_passes=False),
        scratch_types=[
            pltpu.VMEM((P,), jnp.int32),
            pltpu.VMEM((P,), jnp.float32),
            pltpu.VMEM((P,), jnp.float32),
            pltpu.VMEM((P,), jnp.float32),
            pltpu.VMEM((P,), jnp.float32),
            pltpu.VMEM((HW,), jnp.float32),
            pltpu.VMEM((HW,), jnp.float32),
            pltpu.VMEM((P,), jnp.float32),
            pltpu.VMEM((P,), jnp.float32),
            pltpu.SemaphoreType.DMA,
            pltpu.SemaphoreType.DMA,
            pltpu.SemaphoreType.DMA,
            pltpu.SemaphoreType.DMA,
        ],
    )
    return fn(predflat, tgtflat, ipk, w00, w01, w10, w11)


def _cost_body(pred_ref, tgt_ref, minv_ref, mini_ref, num_acc, ssum_acc,
               tsum_acc):
    pj = pl.program_id(1)

    @pl.when(pj == 0)
    def _():
        num_acc[...] = jnp.zeros_like(num_acc)
        ssum_acc[...] = jnp.zeros_like(ssum_acc)
        tsum_acc[...] = jnp.zeros_like(tsum_acc)

    sig = 1.0 / (1.0 + jnp.exp(-pred_ref[0]))
    tgt = tgt_ref[0]
    num_acc[...] += lax.dot_general(
        sig.astype(jnp.bfloat16), tgt.astype(jnp.bfloat16),
        (((1,), (1,)), ((), ())),
        preferred_element_type=jnp.float32)
    s_part = ssum_acc[...]
    t_part = tsum_acc[...]
    for k in range(VREGS_PER_BLK):
        s_part = s_part + sig[:, k * 128:(k + 1) * 128]
        t_part = t_part + tgt[:, k * 128:(k + 1) * 128]
    ssum_acc[...] = s_part
    tsum_acc[...] = t_part

    @pl.when(pj == NPBLK - 1)
    def _():
        num = num_acc[...]
        ssum = jnp.sum(ssum_acc[...], axis=1)
        tsum = jnp.sum(tsum_acc[...], axis=1)
        den = ssum[:, None] + tsum[None, :]
        cost = 1.0 - (2.0 * num + 1.0) / (den + 1.0)
        minv = jnp.min(cost, axis=1)
        iota = lax.broadcasted_iota(jnp.int32, (Q, T), 1)
        mini = jnp.min(jnp.where(cost == minv[:, None], iota, T), axis=1)
        minv_ref[0, 0] = minv
        mini_ref[0, 0] = mini.astype(jnp.int32)


def _cost(sampled_pred, sampled_tgt):
    grid = (B, NPBLK)
    out = pl.pallas_call(
        _cost_body,
        grid=grid,
        in_specs=[
            pl.BlockSpec((1, Q, PBLK), lambda b, pj: (b, 0, pj)),
            pl.BlockSpec((1, T, PBLK), lambda b, pj: (b, 0, pj)),
        ],
        out_specs=[
            pl.BlockSpec((1, 1, Q), lambda b, pj: (b, 0, 0)),
            pl.BlockSpec((1, 1, Q), lambda b, pj: (b, 0, 0)),
        ],
        out_shape=[
            jax.ShapeDtypeStruct((B, 1, Q), jnp.float32),
            jax.ShapeDtypeStruct((B, 1, Q), jnp.int32),
        ],
        scratch_shapes=[
            pltpu.VMEM((Q, T), jnp.float32),
            pltpu.VMEM((Q, 128), jnp.float32),
            pltpu.VMEM((T, 128), jnp.float32),
        ],
        compiler_params=pltpu.CompilerParams(
            dimension_semantics=("arbitrary", "arbitrary")),
    )(sampled_pred, sampled_tgt)
    return out


def kernel(pred_logits, pred_masks, tgt_masks, point_coords):
    del pred_logits
    cx = point_coords[..., 0]
    cy = point_coords[..., 1]
    ipk, w00, w01, w10, w11 = _prep(cx, cy)

    predflat = pred_masks.reshape(B * Q, HW)
    tgtflat = tgt_masks.reshape(B * T, HW)
    sp, st = _sc_sample(predflat, tgtflat, ipk, w00, w01, w10, w11)

    minv, mini = _cost(sp.reshape(B, Q, P), st.reshape(B, T, P))
    minv = minv.reshape(B, Q)
    mini = mini.reshape(B, Q)
    valid = minv < 0.4
    return minv, mini, valid
